# baseline (speedup 1.0000x reference)
import functools

import jax
import jax.numpy as jnp
from jax import lax
from jax.experimental import pallas as pl
from jax.experimental.pallas import tpu as pltpu
from jax.experimental.pallas import tpu_sc as plsc

_N = 10000
_IN = 128
_H = 64
_OUT = 2
_G = 128
_E = 320000

_NC = 2
_NS = 16
_NW = _NC * _NS

_NP = 10016
_RT = _NP // _NS

_CFG = {
    128: dict(C=64, nbuf=2, m0=238, m1=80, e0=240640),
    64: dict(C=128, nbuf=4, m0=128, m1=36, e0=256000),
}


def _make_seg_sum(F):
  mesh = plsc.VectorSubcoreMesh(core_axis_name="c", subcore_axis_name="s")
  cfg = _CFG[F]
  C, nbuf, m0, m1 = cfg['C'], cfg['nbuf'], cfg['m0'], cfg['m1']

  @functools.partial(
      pl.kernel,
      out_type=jax.ShapeDtypeStruct((_NC, _NP, F), jnp.float32),
      mesh=mesh,
      scratch_types=[
          pltpu.VMEM((m0, C), jnp.int32),
          pltpu.VMEM((m0, C), jnp.int32),
          pltpu.VMEM((nbuf, C, F), jnp.float32),
          pltpu.VMEM_SHARED((_NP, F), jnp.float32),
          [pltpu.SemaphoreType.DMA] * nbuf,
      ],
      compiler_params=pltpu.CompilerParams(use_tc_tiling_on_sc=False),
  )
  def seg_sum(h_hbm, srcb_hbm, dstb_hbm, zb_hbm, out_hbm,
              src_v, dst_v, rows_v, acc_sh, sems):
    c = lax.axis_index("c")
    s = lax.axis_index("s")
    m_c = jnp.where(c == 0, m0, m1)
    nk_c = m_c // nbuf
    pltpu.sync_copy(srcb_hbm.at[c, s], src_v)
    pltpu.sync_copy(dstb_hbm.at[c, s], dst_v)
    pltpu.sync_copy(zb_hbm, acc_sh.at[pl.ds(s * _RT, _RT)])
    plsc.subcore_barrier()

    def fire(j, b):
      pltpu.async_copy(h_hbm.at[src_v.at[j]], rows_v.at[b], sems[b])

    def drain(b):
      pltpu.make_async_copy(h_hbm.at[src_v.at[0]], rows_v.at[b], sems[b]).wait()

    def drain_scatter(j, b):
      drain(b)
      pltpu.sync_copy(rows_v.at[b], acc_sh.at[dst_v.at[j]], add=True)

    for b in range(nbuf):
      fire(b, b)

    def group(k, carry):
      for b in range(nbuf):
        j = k * nbuf + b
        drain_scatter(j, b)
        fire(jnp.minimum(j + nbuf, m_c - 1), b)
      return carry

    lax.fori_loop(0, nk_c, group, 0)
    for b in range(nbuf):
      drain(b)
    plsc.subcore_barrier()
    pltpu.sync_copy(acc_sh.at[pl.ds(s * _RT, _RT)],
                    out_hbm.at[c].at[pl.ds(s * _RT, _RT)])

  return seg_sum


_seg_sum_cache = {}


def _seg_sum(F):
  if F not in _seg_sum_cache:
    _seg_sum_cache[F] = _make_seg_sum(F)
  return _seg_sum_cache[F]


def _bn(h, g, b):
  mu = jnp.mean(h, axis=0, keepdims=True)
  var = jnp.mean((h - mu) ** 2, axis=0, keepdims=True)
  return g * (h - mu) / jnp.sqrt(var + 1e-5) + b


def _tc_layer_body(h_ref, agg_ref, w0, b0, g0, be0, w1, b1, g1, be1,
                   w2, b2, og, obeta, out_ref):
  z = h_ref[...] + agg_ref[0, :_N, :] + agg_ref[1, :_N, :]
  h = jnp.dot(z, w0[...], preferred_element_type=jnp.float32) + b0[...]
  h = jnp.maximum(_bn(h, g0[...], be0[...]), 0.0)
  h = jnp.dot(h, w1[...], preferred_element_type=jnp.float32) + b1[...]
  h = jnp.maximum(_bn(h, g1[...], be1[...]), 0.0)
  h = jnp.dot(h, w2[...], preferred_element_type=jnp.float32) + b2[...]
  h = jnp.maximum(_bn(h, og[...], obeta[...]), 0.0)
  out_ref[...] = h


def _tc_pool_body(*refs):
  x_ref, h1_ref, h2_ref, h3_ref, batch_ref = refs[:5]
  wrefs = refs[5:29]
  out_ref = refs[29]
  b = batch_ref[...]
  gid = lax.broadcasted_iota(jnp.int32, (_G, _N), 0)
  p = (gid == b).astype(jnp.float32)
  hiddens = (x_ref[...], h1_ref[...], h2_ref[...], h3_ref[...])
  score = jnp.zeros((_G, _OUT), jnp.float32)
  for i in range(4):
    pooled = jnp.dot(p, hiddens[i], preferred_element_type=jnp.float32)
    w0, b0, w1, b1, w2, b2 = (wrefs[6 * i + j][...] for j in range(6))
    t = jnp.maximum(jnp.dot(pooled, w0, preferred_element_type=jnp.float32) + b0, 0.0)
    t = jnp.maximum(jnp.dot(t, w1, preferred_element_type=jnp.float32) + b1, 0.0)
    score = score + jnp.dot(t, w2, preferred_element_type=jnp.float32) + b2
  out_ref[...] = score


def kernel(x, edge_index, batch, conv_params, outer_bn, mlp_params):
  def pack(F):
    cfg = _CFG[F]
    C, m0, m1, e0 = cfg['C'], cfg['m0'], cfg['m1'], cfg['e0']

    def block(e, m):
      pad = m * C * _NS - e.shape[1]
      sp = jnp.concatenate([e[0], jnp.zeros((pad,), jnp.int32)]).reshape(_NS, m, C)
      dp = jnp.concatenate([e[1], jnp.full((pad,), _N, jnp.int32)]).reshape(_NS, m, C)
      return sp, dp

    s0, d0 = block(edge_index[:, :e0], m0)
    s1, d1 = block(edge_index[:, e0:], m1)
    zpad = ((0, 0), (0, m0 - m1), (0, 0))
    sb = jnp.stack([s0, jnp.pad(s1, zpad)])
    db = jnp.stack([d0, jnp.pad(d1, zpad, constant_values=_N)])
    return sb, db

  packs = {128: pack(128), 64: pack(64)}
  batch2 = batch.reshape(1, _N)

  hidden = [x]
  h = x
  for i in range(3):
    F = _IN if i == 0 else _H
    zb = jnp.zeros((_RT, F), jnp.float32)
    srcb, dstb = packs[F]
    agg = _seg_sum(F)(h, srcb, dstb, zb)
    cp, ob = conv_params[i], outer_bn[i]
    args = (
        h, agg,
        cp['W'][0], cp['b'][0].reshape(1, -1),
        cp['gamma'][0].reshape(1, -1), cp['beta'][0].reshape(1, -1),
        cp['W'][1], cp['b'][1].reshape(1, -1),
        cp['gamma'][1].reshape(1, -1), cp['beta'][1].reshape(1, -1),
        cp['W'][2], cp['b'][2].reshape(1, -1),
        ob['gamma'].reshape(1, -1), ob['beta'].reshape(1, -1),
    )
    h = pl.pallas_call(
        _tc_layer_body,
        out_shape=jax.ShapeDtypeStruct((_N, _H), jnp.float32),
    )(*args)
    hidden.append(h)

  wargs = []
  for i in range(4):
    mp = mlp_params[i]
    for j in range(3):
      wargs += [mp['W'][j], mp['b'][j].reshape(1, -1)]
  score = pl.pallas_call(
      _tc_pool_body,
      out_shape=jax.ShapeDtypeStruct((_G, _OUT), jnp.float32),
  )(hidden[0], hidden[1], hidden[2], hidden[3], batch2, *wargs)
  return score


# speedup vs baseline: 1.0059x; 1.0059x over previous
import functools

import jax
import jax.numpy as jnp
from jax import lax
from jax.experimental import pallas as pl
from jax.experimental.pallas import tpu as pltpu
from jax.experimental.pallas import tpu_sc as plsc

_N = 10000
_IN = 128
_H = 64
_OUT = 2
_G = 128
_E = 320000

_NC = 2
_NS = 16
_NW = _NC * _NS

_NP = 10016
_RT = _NP // _NS

_CFG = {
    128: dict(C=64, nbuf=2, m0=238, m1=80, e0=240640),
    64: dict(C=128, nbuf=4, m0=128, m1=36, e0=256000),
}


def _make_seg_sum(F):
  mesh = plsc.VectorSubcoreMesh(core_axis_name="c", subcore_axis_name="s")
  cfg = _CFG[F]
  C, nbuf, m0, m1 = cfg['C'], cfg['nbuf'], cfg['m0'], cfg['m1']

  @functools.partial(
      pl.kernel,
      out_type=jax.ShapeDtypeStruct((_NC, _NP, F), jnp.float32),
      mesh=mesh,
      scratch_types=[
          pltpu.VMEM((m0, C), jnp.int32),
          pltpu.VMEM((m0, C), jnp.int32),
          pltpu.VMEM((nbuf, C, F), jnp.float32),
          pltpu.VMEM_SHARED((_NP, F), jnp.float32),
          [pltpu.SemaphoreType.DMA] * nbuf,
      ],
      compiler_params=pltpu.CompilerParams(use_tc_tiling_on_sc=False),
  )
  def seg_sum(h_hbm, srcb_hbm, dstb_hbm, zb_hbm, out_hbm,
              src_v, dst_v, rows_v, acc_sh, sems):
    c = lax.axis_index("c")
    s = lax.axis_index("s")
    blk = 1 - c
    m_c = jnp.where(c == 1, m0, m1)
    nk_c = m_c // nbuf
    pltpu.sync_copy(srcb_hbm.at[blk, s], src_v)
    pltpu.sync_copy(dstb_hbm.at[blk, s], dst_v)
    pltpu.sync_copy(zb_hbm, acc_sh.at[pl.ds(s * _RT, _RT)])
    plsc.subcore_barrier()

    def fire(j, b):
      pltpu.async_copy(h_hbm.at[src_v.at[j]], rows_v.at[b], sems[b])

    def drain(b):
      pltpu.make_async_copy(h_hbm.at[src_v.at[0]], rows_v.at[b], sems[b]).wait()

    def drain_scatter(j, b):
      drain(b)
      pltpu.sync_copy(rows_v.at[b], acc_sh.at[dst_v.at[j]], add=True)

    for b in range(nbuf):
      fire(b, b)

    def group(k, carry):
      for b in range(nbuf):
        j = k * nbuf + b
        drain_scatter(j, b)
        fire(jnp.minimum(j + nbuf, m_c - 1), b)
      return carry

    lax.fori_loop(0, nk_c, group, 0)
    for b in range(nbuf):
      drain(b)
    plsc.subcore_barrier()
    pltpu.sync_copy(acc_sh.at[pl.ds(s * _RT, _RT)],
                    out_hbm.at[c].at[pl.ds(s * _RT, _RT)])

  return seg_sum


_seg_sum_cache = {}


def _seg_sum(F):
  if F not in _seg_sum_cache:
    _seg_sum_cache[F] = _make_seg_sum(F)
  return _seg_sum_cache[F]


def _bn(h, g, b):
  mu = jnp.mean(h, axis=0, keepdims=True)
  var = jnp.mean((h - mu) ** 2, axis=0, keepdims=True)
  return g * (h - mu) / jnp.sqrt(var + 1e-5) + b


def _mm(a, w):
  return jnp.dot(a, w, preferred_element_type=jnp.float32)


def _tc_layer_body(h_ref, agg_ref, w0, b0, g0, be0, w1, b1, g1, be1,
                   w2, b2, og, obeta, out_ref):
  z = h_ref[...] + agg_ref[0, :_N, :] + agg_ref[1, :_N, :]
  h = _mm(z, w0[...]) + b0[...]
  h = jnp.maximum(_bn(h, g0[...], be0[...]), 0.0)
  h = _mm(h, w1[...]) + b1[...]
  h = jnp.maximum(_bn(h, g1[...], be1[...]), 0.0)
  h = _mm(h, w2[...]) + b2[...]
  h = jnp.maximum(_bn(h, og[...], obeta[...]), 0.0)
  out_ref[...] = h


def _tc_pool_body(*refs):
  x_ref, h1_ref, h2_ref, h3_ref, batch_ref = refs[:5]
  wrefs = refs[5:29]
  out_ref = refs[29]
  b = batch_ref[...]
  gid = lax.broadcasted_iota(jnp.int32, (_G, _N), 0)
  p = (gid == b).astype(jnp.float32)
  hiddens = (x_ref[...], h1_ref[...], h2_ref[...], h3_ref[...])
  score = jnp.zeros((_G, _OUT), jnp.float32)
  for i in range(4):
    pooled = jnp.dot(p, hiddens[i], preferred_element_type=jnp.float32, precision=lax.Precision.HIGHEST)
    w0, b0, w1, b1, w2, b2 = (wrefs[6 * i + j][...] for j in range(6))
    t = jnp.maximum(_mm(pooled, w0) + b0, 0.0)
    t = jnp.maximum(_mm(t, w1) + b1, 0.0)
    score = score + _mm(t, w2) + b2
  out_ref[...] = score


def kernel(x, edge_index, batch, conv_params, outer_bn, mlp_params):
  def pack(F):
    cfg = _CFG[F]
    C, m0, m1, e0 = cfg['C'], cfg['m0'], cfg['m1'], cfg['e0']

    def block(e, m):
      pad = m * C * _NS - e.shape[1]
      sp = jnp.concatenate([e[0], jnp.zeros((pad,), jnp.int32)]).reshape(_NS, m, C)
      dp = jnp.concatenate([e[1], jnp.full((pad,), _N, jnp.int32)]).reshape(_NS, m, C)
      return sp, dp

    s0, d0 = block(edge_index[:, :e0], m0)
    s1, d1 = block(edge_index[:, e0:], m1)
    zpad = ((0, 0), (0, m0 - m1), (0, 0))
    sb = jnp.stack([s0, jnp.pad(s1, zpad)])
    db = jnp.stack([d0, jnp.pad(d1, zpad, constant_values=_N)])
    return sb, db

  packs = {128: pack(128), 64: pack(64)}
  batch2 = batch.reshape(1, _N)

  hidden = [x]
  h = x
  for i in range(3):
    F = _IN if i == 0 else _H
    zb = jnp.zeros((_RT, F), jnp.float32)
    srcb, dstb = packs[F]
    agg = _seg_sum(F)(h, srcb, dstb, zb)
    cp, ob = conv_params[i], outer_bn[i]
    args = (
        h, agg,
        cp['W'][0], cp['b'][0].reshape(1, -1),
        cp['gamma'][0].reshape(1, -1), cp['beta'][0].reshape(1, -1),
        cp['W'][1], cp['b'][1].reshape(1, -1),
        cp['gamma'][1].reshape(1, -1), cp['beta'][1].reshape(1, -1),
        cp['W'][2], cp['b'][2].reshape(1, -1),
        ob['gamma'].reshape(1, -1), ob['beta'].reshape(1, -1),
    )
    h = pl.pallas_call(
        _tc_layer_body,
        out_shape=jax.ShapeDtypeStruct((_N, _H), jnp.float32),
    )(*args)
    hidden.append(h)

  wargs = []
  for i in range(4):
    mp = mlp_params[i]
    for j in range(3):
      wargs += [mp['W'][j], mp['b'][j].reshape(1, -1)]
  score = pl.pallas_call(
      _tc_pool_body,
      out_shape=jax.ShapeDtypeStruct((_G, _OUT), jnp.float32),
  )(hidden[0], hidden[1], hidden[2], hidden[3], batch2, *wargs)
  return score


# speedup vs baseline: 1.0882x; 1.0818x over previous
import functools

import jax
import jax.numpy as jnp
from jax import lax
from jax.experimental import pallas as pl
from jax.experimental.pallas import tpu as pltpu
from jax.experimental.pallas import tpu_sc as plsc

_N = 10000
_IN = 128
_H = 64
_OUT = 2
_G = 128
_E = 320000

_NC = 2
_NS = 16
_NW = _NC * _NS

_NP = 10016
_RT = _NP // _NS

_CFG = {
    128: dict(C=64, nbuf=2, m0=238, m1=80, e0=240640),
    64: dict(C=128, nbuf=4, m0=128, m1=36, e0=256000),
}


def _make_seg_sum(F):
  mesh = plsc.VectorSubcoreMesh(core_axis_name="c", subcore_axis_name="s")
  cfg = _CFG[F]
  C, nbuf, m0, m1 = cfg['C'], cfg['nbuf'], cfg['m0'], cfg['m1']

  @functools.partial(
      pl.kernel,
      out_type=jax.ShapeDtypeStruct((_NC, _NP, F), jnp.float32),
      mesh=mesh,
      scratch_types=[
          pltpu.VMEM((m0, C), jnp.int32),
          pltpu.VMEM((m0, C), jnp.int32),
          pltpu.VMEM((nbuf, C, F), jnp.float32),
          pltpu.VMEM_SHARED((_NP, F), jnp.float32),
          [pltpu.SemaphoreType.DMA] * nbuf,
      ],
      compiler_params=pltpu.CompilerParams(use_tc_tiling_on_sc=False),
  )
  def seg_sum(h_hbm, srcb_hbm, dstb_hbm, zb_hbm, out_hbm,
              src_v, dst_v, rows_v, acc_sh, sems):
    c = lax.axis_index("c")
    s = lax.axis_index("s")
    pltpu.sync_copy(zb_hbm, acc_sh.at[pl.ds(s * _RT, _RT)])

    def fire(j, b):
      pltpu.async_copy(h_hbm.at[src_v.at[j]], rows_v.at[b], sems[b])

    def drain_scatter(j, b):
      pltpu.make_async_copy(h_hbm.at[src_v.at[0]], rows_v.at[b], sems[b]).wait()
      pltpu.sync_copy(rows_v.at[b], acc_sh.at[dst_v.at[j]], add=True)

    def run(blk, m):
      nk = m // nbuf
      pltpu.sync_copy(srcb_hbm.at[blk, s].at[pl.ds(0, m)], src_v.at[pl.ds(0, m)])
      pltpu.sync_copy(dstb_hbm.at[blk, s].at[pl.ds(0, m)], dst_v.at[pl.ds(0, m)])
      plsc.subcore_barrier()
      for b in range(nbuf):
        fire(b, b)

      def group(k, carry):
        for b in range(nbuf):
          drain_scatter(k * nbuf + b, b)
          fire((k + 1) * nbuf + b, b)
        return carry

      lax.fori_loop(0, nk - 1, group, 0)
      for b in range(nbuf):
        drain_scatter((nk - 1) * nbuf + b, b)

    @pl.when(c == 1)
    def _():
      run(0, m0)

    @pl.when(c == 0)
    def _():
      run(1, m1)

    plsc.subcore_barrier()
    pltpu.sync_copy(acc_sh.at[pl.ds(s * _RT, _RT)],
                    out_hbm.at[c].at[pl.ds(s * _RT, _RT)])

  return seg_sum


_seg_sum_cache = {}


def _seg_sum(F):
  if F not in _seg_sum_cache:
    _seg_sum_cache[F] = _make_seg_sum(F)
  return _seg_sum_cache[F]


def _bn(h, g, b):
  mu = jnp.mean(h, axis=0, keepdims=True)
  var = jnp.mean((h - mu) ** 2, axis=0, keepdims=True)
  return g * (h - mu) / jnp.sqrt(var + 1e-5) + b


def _mm(a, w):
  return jnp.dot(a, w, preferred_element_type=jnp.float32)


def _tc_layer_body(h_ref, agg_ref, w0, b0, g0, be0, w1, b1, g1, be1,
                   w2, b2, og, obeta, out_ref):
  z = h_ref[...] + agg_ref[0, :_N, :] + agg_ref[1, :_N, :]
  h = _mm(z, w0[...]) + b0[...]
  h = jnp.maximum(_bn(h, g0[...], be0[...]), 0.0)
  h = _mm(h, w1[...]) + b1[...]
  h = jnp.maximum(_bn(h, g1[...], be1[...]), 0.0)
  h = _mm(h, w2[...]) + b2[...]
  h = jnp.maximum(_bn(h, og[...], obeta[...]), 0.0)
  out_ref[...] = h


def _tc_pool_body(*refs):
  x_ref, h1_ref, h2_ref, h3_ref, batch_ref = refs[:5]
  wrefs = refs[5:29]
  out_ref = refs[29]
  b = batch_ref[...]
  gid = lax.broadcasted_iota(jnp.int32, (_G, _N), 0)
  p = (gid == b).astype(jnp.float32)
  hiddens = (x_ref[...], h1_ref[...], h2_ref[...], h3_ref[...])
  score = jnp.zeros((_G, _OUT), jnp.float32)
  for i in range(4):
    pooled = jnp.dot(p, hiddens[i], preferred_element_type=jnp.float32, precision=lax.Precision.HIGHEST)
    w0, b0, w1, b1, w2, b2 = (wrefs[6 * i + j][...] for j in range(6))
    t = jnp.maximum(_mm(pooled, w0) + b0, 0.0)
    t = jnp.maximum(_mm(t, w1) + b1, 0.0)
    score = score + _mm(t, w2) + b2
  out_ref[...] = score


def kernel(x, edge_index, batch, conv_params, outer_bn, mlp_params):
  def pack(F):
    cfg = _CFG[F]
    C, m0, m1, e0 = cfg['C'], cfg['m0'], cfg['m1'], cfg['e0']

    def block(e, m):
      pad = m * C * _NS - e.shape[1]
      sp = jnp.concatenate([e[0], jnp.zeros((pad,), jnp.int32)]).reshape(_NS, m, C)
      dp = jnp.concatenate([e[1], jnp.full((pad,), _N, jnp.int32)]).reshape(_NS, m, C)
      return sp, dp

    s0, d0 = block(edge_index[:, :e0], m0)
    s1, d1 = block(edge_index[:, e0:], m1)
    zpad = ((0, 0), (0, m0 - m1), (0, 0))
    sb = jnp.stack([s0, jnp.pad(s1, zpad)])
    db = jnp.stack([d0, jnp.pad(d1, zpad, constant_values=_N)])
    return sb, db

  packs = {128: pack(128), 64: pack(64)}
  batch2 = batch.reshape(1, _N)

  hidden = [x]
  h = x
  for i in range(3):
    F = _IN if i == 0 else _H
    zb = jnp.zeros((_RT, F), jnp.float32)
    srcb, dstb = packs[F]
    agg = _seg_sum(F)(h, srcb, dstb, zb)
    cp, ob = conv_params[i], outer_bn[i]
    args = (
        h, agg,
        cp['W'][0], cp['b'][0].reshape(1, -1),
        cp['gamma'][0].reshape(1, -1), cp['beta'][0].reshape(1, -1),
        cp['W'][1], cp['b'][1].reshape(1, -1),
        cp['gamma'][1].reshape(1, -1), cp['beta'][1].reshape(1, -1),
        cp['W'][2], cp['b'][2].reshape(1, -1),
        ob['gamma'].reshape(1, -1), ob['beta'].reshape(1, -1),
    )
    h = pl.pallas_call(
        _tc_layer_body,
        out_shape=jax.ShapeDtypeStruct((_N, _H), jnp.float32),
    )(*args)
    hidden.append(h)

  wargs = []
  for i in range(4):
    mp = mlp_params[i]
    for j in range(3):
      wargs += [mp['W'][j], mp['b'][j].reshape(1, -1)]
  score = pl.pallas_call(
      _tc_pool_body,
      out_shape=jax.ShapeDtypeStruct((_G, _OUT), jnp.float32),
  )(hidden[0], hidden[1], hidden[2], hidden[3], batch2, *wargs)
  return score


# speedup vs baseline: 1.0988x; 1.0098x over previous
import functools

import jax
import jax.numpy as jnp
from jax import lax
from jax.experimental import pallas as pl
from jax.experimental.pallas import tpu as pltpu
from jax.experimental.pallas import tpu_sc as plsc

_N = 10000
_IN = 128
_H = 64
_OUT = 2
_G = 128
_E = 320000

_NC = 2
_NS = 16
_NW = _NC * _NS

_NP = 10016
_RT = _NP // _NS

_CFG = {
    128: dict(C=64, nbuf=2, m0=238, m1=80, e0=240640),
    64: dict(C=128, nbuf=2, m0=128, m1=36, e0=256000),
}


def _make_seg_sum(F):
  mesh = plsc.VectorSubcoreMesh(core_axis_name="c", subcore_axis_name="s")
  cfg = _CFG[F]
  C, nbuf, m0, m1 = cfg['C'], cfg['nbuf'], cfg['m0'], cfg['m1']

  @functools.partial(
      pl.kernel,
      out_type=jax.ShapeDtypeStruct((_NC, _NP, F), jnp.float32),
      mesh=mesh,
      scratch_types=[
          pltpu.VMEM((m0, C), jnp.int32),
          pltpu.VMEM((m0, C), jnp.int32),
          pltpu.VMEM((nbuf, C, F), jnp.float32),
          pltpu.VMEM_SHARED((_NP, F), jnp.float32),
          [pltpu.SemaphoreType.DMA] * nbuf,
      ],
      compiler_params=pltpu.CompilerParams(use_tc_tiling_on_sc=False),
  )
  def seg_sum(h_hbm, srcb_hbm, dstb_hbm, zb_hbm, out_hbm,
              src_v, dst_v, rows_v, acc_sh, sems):
    c = lax.axis_index("c")
    s = lax.axis_index("s")
    pltpu.sync_copy(zb_hbm, acc_sh.at[pl.ds(s * _RT, _RT)])

    def fire(j, b):
      pltpu.async_copy(h_hbm.at[src_v.at[j]], rows_v.at[b], sems[b])

    def drain_scatter(j, b):
      pltpu.make_async_copy(h_hbm.at[src_v.at[0]], rows_v.at[b], sems[b]).wait()
      pltpu.sync_copy(rows_v.at[b], acc_sh.at[dst_v.at[j]], add=True)

    def run(blk, m):
      nk = m // nbuf
      pltpu.sync_copy(srcb_hbm.at[blk, s].at[pl.ds(0, m)], src_v.at[pl.ds(0, m)])
      pltpu.sync_copy(dstb_hbm.at[blk, s].at[pl.ds(0, m)], dst_v.at[pl.ds(0, m)])
      plsc.subcore_barrier()
      for b in range(nbuf):
        fire(b, b)

      def group(k, carry):
        for b in range(nbuf):
          drain_scatter(k * nbuf + b, b)
          fire((k + 1) * nbuf + b, b)
        return carry

      lax.fori_loop(0, nk - 1, group, 0)
      for b in range(nbuf):
        drain_scatter((nk - 1) * nbuf + b, b)

    @pl.when(c == 1)
    def _():
      run(0, m0)

    @pl.when(c == 0)
    def _():
      run(1, m1)

    plsc.subcore_barrier()
    pltpu.sync_copy(acc_sh.at[pl.ds(s * _RT, _RT)],
                    out_hbm.at[c].at[pl.ds(s * _RT, _RT)])

  return seg_sum


_seg_sum_cache = {}


def _seg_sum(F):
  if F not in _seg_sum_cache:
    _seg_sum_cache[F] = _make_seg_sum(F)
  return _seg_sum_cache[F]


def _bn(h, g, b):
  mu = jnp.mean(h, axis=0, keepdims=True)
  var = jnp.mean((h - mu) ** 2, axis=0, keepdims=True)
  return g * (h - mu) / jnp.sqrt(var + 1e-5) + b


def _mm(a, w):
  return jnp.dot(a, w, preferred_element_type=jnp.float32)


def _tc_layer_body(h_ref, agg_ref, w0, b0, g0, be0, w1, b1, g1, be1,
                   w2, b2, og, obeta, out_ref):
  z = h_ref[...] + agg_ref[0, :_N, :] + agg_ref[1, :_N, :]
  h = _mm(z, w0[...]) + b0[...]
  h = jnp.maximum(_bn(h, g0[...], be0[...]), 0.0)
  h = _mm(h, w1[...]) + b1[...]
  h = jnp.maximum(_bn(h, g1[...], be1[...]), 0.0)
  h = _mm(h, w2[...]) + b2[...]
  h = jnp.maximum(_bn(h, og[...], obeta[...]), 0.0)
  out_ref[...] = h


def _tc_pool_body(*refs):
  x_ref, h1_ref, h2_ref, h3_ref, batch_ref = refs[:5]
  wrefs = refs[5:29]
  out_ref = refs[29]
  b = batch_ref[...]
  gid = lax.broadcasted_iota(jnp.int32, (_G, _N), 0)
  p = (gid == b).astype(jnp.float32)
  hiddens = (x_ref[...], h1_ref[...], h2_ref[...], h3_ref[...])
  score = jnp.zeros((_G, _OUT), jnp.float32)
  for i in range(4):
    pooled = jnp.dot(p, hiddens[i], preferred_element_type=jnp.float32, precision=lax.Precision.HIGHEST)
    w0, b0, w1, b1, w2, b2 = (wrefs[6 * i + j][...] for j in range(6))
    t = jnp.maximum(_mm(pooled, w0) + b0, 0.0)
    t = jnp.maximum(_mm(t, w1) + b1, 0.0)
    score = score + _mm(t, w2) + b2
  out_ref[...] = score


def kernel(x, edge_index, batch, conv_params, outer_bn, mlp_params):
  def pack(F):
    cfg = _CFG[F]
    C, m0, m1, e0 = cfg['C'], cfg['m0'], cfg['m1'], cfg['e0']

    def block(e, m):
      pad = m * C * _NS - e.shape[1]
      sp = jnp.concatenate([e[0], jnp.zeros((pad,), jnp.int32)]).reshape(_NS, m, C)
      dp = jnp.concatenate([e[1], jnp.full((pad,), _N, jnp.int32)]).reshape(_NS, m, C)
      return sp, dp

    s0, d0 = block(edge_index[:, :e0], m0)
    s1, d1 = block(edge_index[:, e0:], m1)
    zpad = ((0, 0), (0, m0 - m1), (0, 0))
    sb = jnp.stack([s0, jnp.pad(s1, zpad)])
    db = jnp.stack([d0, jnp.pad(d1, zpad, constant_values=_N)])
    return sb, db

  packs = {128: pack(128), 64: pack(64)}
  batch2 = batch.reshape(1, _N)

  hidden = [x]
  h = x
  for i in range(3):
    F = _IN if i == 0 else _H
    zb = jnp.zeros((_RT, F), jnp.float32)
    srcb, dstb = packs[F]
    agg = _seg_sum(F)(h, srcb, dstb, zb)
    cp, ob = conv_params[i], outer_bn[i]
    args = (
        h, agg,
        cp['W'][0], cp['b'][0].reshape(1, -1),
        cp['gamma'][0].reshape(1, -1), cp['beta'][0].reshape(1, -1),
        cp['W'][1], cp['b'][1].reshape(1, -1),
        cp['gamma'][1].reshape(1, -1), cp['beta'][1].reshape(1, -1),
        cp['W'][2], cp['b'][2].reshape(1, -1),
        ob['gamma'].reshape(1, -1), ob['beta'].reshape(1, -1),
    )
    h = pl.pallas_call(
        _tc_layer_body,
        out_shape=jax.ShapeDtypeStruct((_N, _H), jnp.float32),
    )(*args)
    hidden.append(h)

  wargs = []
  for i in range(4):
    mp = mlp_params[i]
    for j in range(3):
      wargs += [mp['W'][j], mp['b'][j].reshape(1, -1)]
  score = pl.pallas_call(
      _tc_pool_body,
      out_shape=jax.ShapeDtypeStruct((_G, _OUT), jnp.float32),
  )(hidden[0], hidden[1], hidden[2], hidden[3], batch2, *wargs)
  return score


# speedup vs baseline: 1.4909x; 1.3568x over previous
import functools

import jax
import jax.numpy as jnp
from jax import lax
from jax.experimental import pallas as pl
from jax.experimental.pallas import tpu as pltpu
from jax.experimental.pallas import tpu_sc as plsc

_N = 10000
_IN = 128
_H = 64
_OUT = 2
_G = 128
_E = 320000

_NC = 2
_NS = 16
_NW = _NC * _NS

_NP = 10016
_RT = _NP // _NS

_C = 128
_NCH = -(-_E // (_NW * _C))
_EPW = _NCH * _C
_EP = _EPW * _NW


def _make_seg_sum(F):
  mesh = plsc.VectorSubcoreMesh(core_axis_name="c", subcore_axis_name="s")

  @functools.partial(
      pl.kernel,
      out_type=jax.ShapeDtypeStruct((_NC, _NP, F), jnp.float32),
      mesh=mesh,
      scratch_types=[
          pltpu.VMEM((_NCH, _C), jnp.int32),
          pltpu.VMEM((_NCH, _C), jnp.int32),
          pltpu.VMEM((_C, F), jnp.float32),
          pltpu.VMEM_SHARED((_NP, F), jnp.float32),
          pltpu.SemaphoreType.DMA,
      ],
      compiler_params=pltpu.CompilerParams(use_tc_tiling_on_sc=False),
  )
  def seg_sum(h_hbm, srcb_hbm, dstb_hbm, zb_hbm, out_hbm,
              src_v, dst_v, rows_v, acc_sh, sem):
    c = lax.axis_index("c")
    s = lax.axis_index("s")
    wid = s * _NC + c
    pltpu.sync_copy(srcb_hbm.at[wid], src_v)
    pltpu.sync_copy(dstb_hbm.at[wid], dst_v)
    pltpu.sync_copy(zb_hbm, acc_sh.at[pl.ds(s * _RT, _RT)])
    plsc.subcore_barrier()

    def chunk(j, carry):
      pltpu.async_copy(h_hbm.at[src_v.at[j]], rows_v, sem).wait()
      pltpu.sync_copy(rows_v, acc_sh.at[dst_v.at[j]], add=True)
      return carry

    lax.fori_loop(0, _NCH, chunk, 0)
    plsc.subcore_barrier()
    pltpu.sync_copy(acc_sh.at[pl.ds(s * _RT, _RT)],
                    out_hbm.at[c].at[pl.ds(s * _RT, _RT)])

  return seg_sum


_seg_sum_cache = {}


def _seg_sum(F):
  if F not in _seg_sum_cache:
    _seg_sum_cache[F] = _make_seg_sum(F)
  return _seg_sum_cache[F]


def _bn(h, g, b):
  mu = jnp.mean(h, axis=0, keepdims=True)
  var = jnp.mean((h - mu) ** 2, axis=0, keepdims=True)
  return g * (h - mu) / jnp.sqrt(var + 1e-5) + b


def _mm(a, w):
  return jnp.dot(a, w, preferred_element_type=jnp.float32)


def _tc_layer_body(h_ref, agg_ref, w0, b0, g0, be0, w1, b1, g1, be1,
                   w2, b2, og, obeta, out_ref):
  z = h_ref[...] + agg_ref[0, :_N, :] + agg_ref[1, :_N, :]
  h = _mm(z, w0[...]) + b0[...]
  h = jnp.maximum(_bn(h, g0[...], be0[...]), 0.0)
  h = _mm(h, w1[...]) + b1[...]
  h = jnp.maximum(_bn(h, g1[...], be1[...]), 0.0)
  h = _mm(h, w2[...]) + b2[...]
  h = jnp.maximum(_bn(h, og[...], obeta[...]), 0.0)
  out_ref[...] = h


def _tc_pool_body(*refs):
  x_ref, h1_ref, h2_ref, h3_ref, batch_ref = refs[:5]
  wrefs = refs[5:29]
  out_ref = refs[29]
  b = batch_ref[...]
  gid = lax.broadcasted_iota(jnp.int32, (_G, _N), 0)
  p = (gid == b).astype(jnp.float32)
  hiddens = (x_ref[...], h1_ref[...], h2_ref[...], h3_ref[...])
  score = jnp.zeros((_G, _OUT), jnp.float32)
  for i in range(4):
    pooled = jnp.dot(p, hiddens[i], preferred_element_type=jnp.float32, precision=lax.Precision.HIGHEST)
    w0, b0, w1, b1, w2, b2 = (wrefs[6 * i + j][...] for j in range(6))
    t = jnp.maximum(_mm(pooled, w0) + b0, 0.0)
    t = jnp.maximum(_mm(t, w1) + b1, 0.0)
    score = score + _mm(t, w2) + b2
  out_ref[...] = score


def kernel(x, edge_index, batch, conv_params, outer_bn, mlp_params):
  def pack():
    pad = _EP - _E
    sb = jnp.concatenate([edge_index[0], jnp.zeros((pad,), jnp.int32)]
                         ).reshape(_NW, _NCH, _C)
    db = jnp.concatenate([edge_index[1], jnp.full((pad,), _N, jnp.int32)]
                         ).reshape(_NW, _NCH, _C)
    return sb, db

  srcb, dstb = pack()
  batch2 = batch.reshape(1, _N)

  hidden = [x]
  h = x
  for i in range(3):
    F = _IN if i == 0 else _H
    zb = jnp.zeros((_RT, F), jnp.float32)
    agg = _seg_sum(F)(h, srcb, dstb, zb)
    cp, ob = conv_params[i], outer_bn[i]
    args = (
        h, agg,
        cp['W'][0], cp['b'][0].reshape(1, -1),
        cp['gamma'][0].reshape(1, -1), cp['beta'][0].reshape(1, -1),
        cp['W'][1], cp['b'][1].reshape(1, -1),
        cp['gamma'][1].reshape(1, -1), cp['beta'][1].reshape(1, -1),
        cp['W'][2], cp['b'][2].reshape(1, -1),
        ob['gamma'].reshape(1, -1), ob['beta'].reshape(1, -1),
    )
    h = pl.pallas_call(
        _tc_layer_body,
        out_shape=jax.ShapeDtypeStruct((_N, _H), jnp.float32),
    )(*args)
    hidden.append(h)

  wargs = []
  for i in range(4):
    mp = mlp_params[i]
    for j in range(3):
      wargs += [mp['W'][j], mp['b'][j].reshape(1, -1)]
  score = pl.pallas_call(
      _tc_pool_body,
      out_shape=jax.ShapeDtypeStruct((_G, _OUT), jnp.float32),
  )(hidden[0], hidden[1], hidden[2], hidden[3], batch2, *wargs)
  return score
